# baseline (speedup 1.0000x reference)
import functools

import jax
import jax.numpy as jnp
from jax import lax
from jax.experimental import pallas as pl
from jax.experimental.pallas import tpu as pltpu
from jax.experimental.pallas import tpu_sc as plsc

_BLOCK_COLS = 49152


def _sc_lookup_body(ct_hbm, table_hbm, patt_hbm, ct_v, ttab_v, patt_v):
    c = lax.axis_index("c")
    s = lax.axis_index("s")

    @pl.when(jnp.logical_and(c == 0, s == 0))
    def _():
        pltpu.sync_copy(ct_hbm, ct_v)
        pltpu.sync_copy(table_hbm, ttab_v)
        ct16 = plsc.load_gather(ct_v, [jnp.zeros((16,), jnp.int32)])
        for j in range(64):
            v = plsc.load_gather(ttab_v, [ct16, jnp.full((16,), j, jnp.int32)])
            for l in range(8):
                patt_v[j, pl.ds(16 * l, 16)] = v
        pltpu.sync_copy(patt_v, patt_hbm)


def _tc_body(id_ref, tt_ref, x_ref, o_ref):
    ct = id_ref[0]
    tt = tt_ref[...]
    lane = jax.lax.broadcasted_iota(jnp.int32, tt.shape, 1)
    col = jnp.sum(jnp.where(lane == ct, tt, 0.0), axis=1, keepdims=True)
    o_ref[...] = x_ref[...] + col


def kernel(x, cell_type_id, table):
    n, d = x.shape
    xt = x.T
    tt = table.T
    ct = cell_type_id.astype(jnp.int32)

    mesh = plsc.VectorSubcoreMesh(core_axis_name="c", subcore_axis_name="s")
    sc_lookup = functools.partial(
        pl.kernel,
        out_type=jax.ShapeDtypeStruct((d, 128), jnp.float32),
        mesh=mesh,
        scratch_types=[
            pltpu.VMEM((1,), jnp.int32),
            pltpu.VMEM(table.shape, jnp.float32),
            pltpu.VMEM((d, 128), jnp.float32),
        ],
        compiler_params=pltpu.CompilerParams(
            needs_layout_passes=False, skip_device_barrier=True
        ),
    )(_sc_lookup_body)
    patt = sc_lookup(ct, table)

    grid = pl.cdiv(n, _BLOCK_COLS)
    outt = pl.pallas_call(
        _tc_body,
        grid=(grid,),
        in_specs=[
            pl.BlockSpec(memory_space=pltpu.SMEM),
            pl.BlockSpec((d, tt.shape[1]), lambda i: (0, 0)),
            pl.BlockSpec((d, _BLOCK_COLS), lambda i: (0, i)),
        ],
        out_specs=pl.BlockSpec((d, _BLOCK_COLS), lambda i: (0, i)),
        out_shape=jax.ShapeDtypeStruct((d, n), jnp.float32),
        compiler_params=pltpu.CompilerParams(
            dimension_semantics=("parallel",),
            skip_device_barrier=True,
        ),
    )(ct, tt, xt)

    tail = xt[:, n - 128 :] + patt[:, 0:1]
    outt = lax.dynamic_update_slice(outt, tail, (0, n - 128))
    return outt.T


# speedup vs baseline: 1.5830x; 1.5830x over previous
import jax
import jax.numpy as jnp
from jax.experimental import pallas as pl
from jax.experimental.pallas import tpu as pltpu

_BLOCK_COLS = 49152


def _tc_body(id_ref, tt_ref, x_ref, o_ref):
    ct = id_ref[0]
    tt = tt_ref[...]
    lane = jax.lax.broadcasted_iota(jnp.int32, tt.shape, 1)
    col = jnp.sum(jnp.where(lane == ct, tt, 0.0), axis=1, keepdims=True)
    o_ref[...] = x_ref[...] + col


def kernel(x, cell_type_id, table):
    n, d = x.shape
    xt = x.T
    tt = table.T
    ct = cell_type_id.astype(jnp.int32)
    grid = pl.cdiv(n, _BLOCK_COLS)

    outt = pl.pallas_call(
        _tc_body,
        grid=(grid,),
        in_specs=[
            pl.BlockSpec(memory_space=pltpu.SMEM),
            pl.BlockSpec((d, tt.shape[1]), lambda i: (0, 0)),
            pl.BlockSpec((d, _BLOCK_COLS), lambda i: (0, i)),
        ],
        out_specs=pl.BlockSpec((d, _BLOCK_COLS), lambda i: (0, i)),
        out_shape=jax.ShapeDtypeStruct((d, n), jnp.float32),
        compiler_params=pltpu.CompilerParams(
            dimension_semantics=("parallel",),
        ),
    )(ct, tt, xt)
    return outt.T
